# baseline (speedup 1.0000x reference)
import functools

import jax
import jax.numpy as jnp
from jax import lax
from jax.experimental import pallas as pl
from jax.experimental.pallas import tpu as pltpu
from jax.experimental.pallas import tpu_sc as plsc

_N_EMBED = 1024
_NUM_EXPERTS = 8
_TOP_K = 2
_BM = 256
_SL = 8


def _router(x, Wg, bg, Wn, bn):
    logits = x @ Wg + bg
    safe_noise = jax.nn.softplus(x @ Wn + bn)
    eps = jax.random.normal(jax.random.key(42), logits.shape, dtype=logits.dtype)
    noisy_logits = logits + eps * safe_noise
    top_v, top_i = lax.top_k(noisy_logits, _TOP_K)
    probs = jax.nn.softmax(top_v, axis=-1)
    return top_i, probs


def _dispatch(top_i, probs, n_tokens):
    E, BM, K = _NUM_EXPERTS, _BM, _TOP_K
    A = n_tokens * K
    P = A + E * BM

    flat_e = top_i.reshape(-1).astype(jnp.int32)
    oh = jax.nn.one_hot(flat_e, E, dtype=jnp.int32)
    csum = jnp.cumsum(oh, axis=0)
    counts = csum[-1]
    rank = jnp.take_along_axis(csum, flat_e[:, None], axis=1)[:, 0] - 1
    padded = ((counts + BM - 1) // BM) * BM
    pstart = jnp.concatenate([jnp.zeros((1,), jnp.int32),
                              jnp.cumsum(padded)[:-1].astype(jnp.int32)])
    dst = pstart[flat_e] + rank
    src_assign = jnp.zeros((P,), jnp.int32).at[dst].set(
        jnp.arange(A, dtype=jnp.int32))
    valid = jnp.zeros((P,), jnp.bool_).at[dst].set(True)
    src_token = src_assign // K
    ws = jnp.where(valid, probs.reshape(-1)[src_assign], 0.0)
    n_m = P // BM
    gids = (jnp.searchsorted(pstart,
                             jnp.arange(n_m, dtype=jnp.int32) * BM,
                             side="right").astype(jnp.int32) - 1)
    pos = dst.reshape(n_tokens, K)
    return src_token, ws, gids, pos, P


def _sc_gather(table, idx):
    P = idx.shape[0]
    D2 = table.shape[1]
    info = plsc.get_sparse_core_info()
    NW = info.num_cores * info.num_subcores
    rows_per_w = P // NW
    CH = 48
    n_ch = rows_per_w // CH
    assert n_ch * CH == rows_per_w
    mesh = plsc.VectorSubcoreMesh(core_axis_name="c", subcore_axis_name="s")

    @functools.partial(
        pl.kernel,
        mesh=mesh,
        out_type=jax.ShapeDtypeStruct((P, D2), jnp.float32),
        scratch_types=[
            pltpu.VMEM((CH,), jnp.int32),
            pltpu.VMEM((CH,), jnp.int32),
            pltpu.VMEM((CH, D2), jnp.float32),
            pltpu.VMEM((CH, D2), jnp.float32),
            pltpu.SemaphoreType.DMA,
            pltpu.SemaphoreType.DMA,
        ],
    )
    def k(table_hbm, idx_hbm, out_hbm, idx0, idx1, buf0, buf1, sg0, sg1):
        wid = lax.axis_index("s") * info.num_cores + lax.axis_index("c")
        base = wid * rows_per_w
        idxb, bufb, sg = [idx0, idx1], [buf0, buf1], [sg0, sg1]

        pltpu.sync_copy(idx_hbm.at[pl.ds(base, CH)], idx0)
        gathers = [pltpu.async_copy(table_hbm.at[idx0], buf0, sg0), None]
        for i in range(n_ch):
            b = i & 1
            nb = 1 - b
            if i + 1 < n_ch:
                off = base + (i + 1) * CH
                pltpu.sync_copy(idx_hbm.at[pl.ds(off, CH)], idxb[nb])
                gathers[nb] = pltpu.async_copy(
                    table_hbm.at[idxb[nb]], bufb[nb], sg[nb])
            gathers[b].wait()
            pltpu.sync_copy(bufb[b], out_hbm.at[pl.ds(base + i * CH, CH)])

    return k(table, idx)


def _sc_combine(ys, pos0, pos1):
    N = pos0.shape[0]
    D2 = ys.shape[1]
    info = plsc.get_sparse_core_info()
    NW = info.num_cores * info.num_subcores
    rows_per_w = N // NW
    CH = 16
    n_ch = rows_per_w // CH
    assert n_ch * CH == rows_per_w
    mesh = plsc.VectorSubcoreMesh(core_axis_name="c", subcore_axis_name="s")

    @functools.partial(
        pl.kernel,
        mesh=mesh,
        out_type=jax.ShapeDtypeStruct((N, D2), jnp.float32),
        scratch_types=[
            pltpu.VMEM((CH,), jnp.int32),
            pltpu.VMEM((CH,), jnp.int32),
            pltpu.VMEM((CH, D2), jnp.float32),
            pltpu.VMEM((CH, D2), jnp.float32),
            pltpu.VMEM((CH, D2), jnp.float32),
            pltpu.VMEM((CH, D2), jnp.float32),
            pltpu.SemaphoreType.DMA,
            pltpu.SemaphoreType.DMA,
            pltpu.SemaphoreType.DMA,
            pltpu.SemaphoreType.DMA,
        ],
    )
    def k(ys_hbm, p0_hbm, p1_hbm, out_hbm,
          i0, i1, a0, a1, b0, b1, sa0, sa1, sb0, sb1):
        wid = lax.axis_index("s") * info.num_cores + lax.axis_index("c")
        base = wid * rows_per_w
        idxb, ab, bb = [i0, i1], [a0, a1], [b0, b1]
        sa, sb = [sa0, sa1], [sb0, sb1]

        def issue(slot, off):
            pltpu.sync_copy(p0_hbm.at[pl.ds(off, CH)], idxb[slot])
            ga = pltpu.async_copy(ys_hbm.at[idxb[slot]], ab[slot], sa[slot])
            pltpu.sync_copy(p1_hbm.at[pl.ds(off, CH)], idxb[slot])
            gb = pltpu.async_copy(ys_hbm.at[idxb[slot]], bb[slot], sb[slot])
            return ga, gb

        gathers = [issue(0, base), None]
        for i in range(n_ch):
            b = i & 1
            nb = 1 - b
            if i + 1 < n_ch:
                gathers[nb] = issue(nb, base + (i + 1) * CH)
            ga, gb = gathers[b]
            ga.wait()
            gb.wait()
            a_v, b_v = ab[b], bb[b]

            def add_row(r, c1):
                def add_c(c, c2):
                    sl = pl.ds(c * 16, 16)
                    a_v[r, sl] = a_v[r, sl] + b_v[r, sl]
                    return c2
                return lax.fori_loop(0, D2 // 16, add_c, c1, unroll=8)

            lax.fori_loop(0, CH, add_row, 0)
            pltpu.sync_copy(a_v, out_hbm.at[pl.ds(base + i * CH, CH)])

    return k(ys, pos0, pos1)


def _ffn(gids, xs, W1b, b1r, W2b, b2r, ws):
    P, D = xs.shape
    E, _, H = W1b.shape
    n_m = P // _BM

    def kern(g_ref, xs_ref, w1_ref, b1_ref, w2_ref, b2_ref, ws_ref, o_ref):
        xb = xs_ref[...].astype(jnp.bfloat16)
        h = jnp.dot(xb, w1_ref[0], preferred_element_type=jnp.float32)
        h = jnp.maximum(h + b1_ref[0], 0.0).astype(jnp.bfloat16)
        acc = jnp.dot(h, w2_ref[0], preferred_element_type=jnp.float32)
        o_ref[...] = (acc + b2_ref[0]) * ws_ref[...]

    grid_spec = pltpu.PrefetchScalarGridSpec(
        num_scalar_prefetch=1,
        grid=(n_m,),
        in_specs=[
            pl.BlockSpec((_BM, D), lambda i, g: (i, 0)),
            pl.BlockSpec((1, D, H), lambda i, g: (g[i], 0, 0)),
            pl.BlockSpec((1, 1, H), lambda i, g: (g[i], 0, 0)),
            pl.BlockSpec((1, H, D), lambda i, g: (g[i], 0, 0)),
            pl.BlockSpec((1, 1, D), lambda i, g: (g[i], 0, 0)),
            pl.BlockSpec((_BM, 1), lambda i, g: (i, 0)),
        ],
        out_specs=pl.BlockSpec((_BM, D), lambda i, g: (i, 0)),
    )
    return pl.pallas_call(
        kern,
        grid_spec=grid_spec,
        out_shape=jax.ShapeDtypeStruct((P, D), jnp.float32),
        compiler_params=pltpu.CompilerParams(
            dimension_semantics=("arbitrary",)),
    )(gids, xs, W1b, b1r, W2b, b2r, ws)


def kernel(x, Wg, bg, Wn, bn, W1, b1, W2, b2):
    B, S, D = x.shape
    N = B * S
    E, _, H = W1.shape
    top_i, probs = _router(x, Wg, bg, Wn, bn)
    src_token, ws, gids, pos, P = _dispatch(top_i, probs, N)

    x2d = x.reshape(N, D)
    P2 = P // 2
    nm2 = P2 // _BM
    W1b = W1.astype(jnp.bfloat16)
    W2b = W2.astype(jnp.bfloat16)
    b1r = b1.reshape(E, 1, H)
    b2r = b2.reshape(E, 1, D)
    ws2 = ws.reshape(P, 1)
    ys_halves = []
    for hh in range(2):
        sl = slice(hh * P2, (hh + 1) * P2)
        xs_h = _sc_gather(x2d, src_token[sl])
        ys_halves.append(
            _ffn(gids[hh * nm2:(hh + 1) * nm2], xs_h,
                 W1b, b1r, W2b, b2r, ws2[sl]))
    ys = jnp.concatenate(ys_halves, axis=0)
    out = _sc_combine(ys, pos[:, 0], pos[:, 1])
    return out.reshape(B, S, D)


# speedup vs baseline: 1.0478x; 1.0478x over previous
import functools

import jax
import jax.numpy as jnp
from jax import lax
from jax.experimental import pallas as pl
from jax.experimental.pallas import tpu as pltpu
from jax.experimental.pallas import tpu_sc as plsc

_N_EMBED = 1024
_NUM_EXPERTS = 8
_TOP_K = 2
_BM = 256
_SL = 8


def _router(x, Wg, bg, Wn, bn):
    logits = x @ Wg + bg
    safe_noise = jax.nn.softplus(x @ Wn + bn)
    eps = jax.random.normal(jax.random.key(42), logits.shape, dtype=logits.dtype)
    noisy_logits = logits + eps * safe_noise
    top_v, top_i = lax.top_k(noisy_logits, _TOP_K)
    probs = jax.nn.softmax(top_v, axis=-1)
    return top_i, probs


def _dispatch(top_i, probs, n_tokens):
    E, BM, K = _NUM_EXPERTS, _BM, _TOP_K
    A = n_tokens * K
    P = A + E * BM

    flat_e = top_i.reshape(-1).astype(jnp.int32)
    oh = jax.nn.one_hot(flat_e, E, dtype=jnp.int32)
    csum = jnp.cumsum(oh, axis=0)
    counts = csum[-1]
    rank = jnp.take_along_axis(csum, flat_e[:, None], axis=1)[:, 0] - 1
    padded = ((counts + BM - 1) // BM) * BM
    pstart = jnp.concatenate([jnp.zeros((1,), jnp.int32),
                              jnp.cumsum(padded)[:-1].astype(jnp.int32)])
    dst = pstart[flat_e] + rank
    src_assign = jnp.zeros((P,), jnp.int32).at[dst].set(
        jnp.arange(A, dtype=jnp.int32))
    valid = jnp.zeros((P,), jnp.bool_).at[dst].set(True)
    src_token = src_assign // K
    ws = jnp.where(valid, probs.reshape(-1)[src_assign], 0.0)
    n_m = P // BM
    gids = (jnp.searchsorted(pstart,
                             jnp.arange(n_m, dtype=jnp.int32) * BM,
                             side="right").astype(jnp.int32) - 1)
    pos = dst.reshape(n_tokens, K)
    return src_token, ws, gids, pos, P


def _sc_gather(table, idx):
    P = idx.shape[0]
    D2 = table.shape[1]
    info = plsc.get_sparse_core_info()
    NW = info.num_cores * info.num_subcores
    rows_per_w = P // NW
    CH = 48
    n_ch = rows_per_w // CH
    assert n_ch * CH == rows_per_w
    mesh = plsc.VectorSubcoreMesh(core_axis_name="c", subcore_axis_name="s")

    @functools.partial(
        pl.kernel,
        mesh=mesh,
        out_type=jax.ShapeDtypeStruct((P, D2), jnp.float32),
        scratch_types=[
            pltpu.VMEM((CH,), jnp.int32),
            pltpu.VMEM((CH,), jnp.int32),
            pltpu.VMEM((CH, D2), jnp.float32),
            pltpu.VMEM((CH, D2), jnp.float32),
            pltpu.SemaphoreType.DMA,
            pltpu.SemaphoreType.DMA,
        ],
    )
    def k(table_hbm, idx_hbm, out_hbm, idx0, idx1, buf0, buf1, sg0, sg1):
        wid = lax.axis_index("s") * info.num_cores + lax.axis_index("c")
        base = wid * rows_per_w
        idxb, bufb, sg = [idx0, idx1], [buf0, buf1], [sg0, sg1]

        pltpu.sync_copy(idx_hbm.at[pl.ds(base, CH)], idx0)
        gathers = [pltpu.async_copy(table_hbm.at[idx0], buf0, sg0), None]
        for i in range(n_ch):
            b = i & 1
            nb = 1 - b
            if i + 1 < n_ch:
                off = base + (i + 1) * CH
                pltpu.sync_copy(idx_hbm.at[pl.ds(off, CH)], idxb[nb])
                gathers[nb] = pltpu.async_copy(
                    table_hbm.at[idxb[nb]], bufb[nb], sg[nb])
            gathers[b].wait()
            pltpu.sync_copy(bufb[b], out_hbm.at[pl.ds(base + i * CH, CH)])

    return k(table, idx)


def _sc_combine(ys, pos0, pos1):
    N = pos0.shape[0]
    D2 = ys.shape[1]
    info = plsc.get_sparse_core_info()
    NW = info.num_cores * info.num_subcores
    rows_per_w = N // NW
    CH = 16
    n_ch = rows_per_w // CH
    assert n_ch * CH == rows_per_w
    mesh = plsc.VectorSubcoreMesh(core_axis_name="c", subcore_axis_name="s")

    @functools.partial(
        pl.kernel,
        mesh=mesh,
        out_type=jax.ShapeDtypeStruct((N, D2), jnp.float32),
        scratch_types=[
            pltpu.VMEM((CH,), jnp.int32),
            pltpu.VMEM((CH,), jnp.int32),
            pltpu.VMEM((CH, D2), jnp.float32),
            pltpu.VMEM((CH, D2), jnp.float32),
            pltpu.VMEM((CH, D2), jnp.float32),
            pltpu.VMEM((CH, D2), jnp.float32),
            pltpu.SemaphoreType.DMA,
            pltpu.SemaphoreType.DMA,
            pltpu.SemaphoreType.DMA,
            pltpu.SemaphoreType.DMA,
        ],
    )
    def k(ys_hbm, p0_hbm, p1_hbm, out_hbm,
          i0, i1, a0, a1, b0, b1, sa0, sa1, sb0, sb1):
        wid = lax.axis_index("s") * info.num_cores + lax.axis_index("c")
        base = wid * rows_per_w
        idxb, ab, bb = [i0, i1], [a0, a1], [b0, b1]
        sa, sb = [sa0, sa1], [sb0, sb1]

        def issue(slot, off):
            pltpu.sync_copy(p0_hbm.at[pl.ds(off, CH)], idxb[slot])
            ga = pltpu.async_copy(ys_hbm.at[idxb[slot]], ab[slot], sa[slot])
            pltpu.sync_copy(p1_hbm.at[pl.ds(off, CH)], idxb[slot])
            gb = pltpu.async_copy(ys_hbm.at[idxb[slot]], bb[slot], sb[slot])
            return ga, gb

        gathers = [issue(0, base), None]
        for i in range(n_ch):
            b = i & 1
            nb = 1 - b
            if i + 1 < n_ch:
                gathers[nb] = issue(nb, base + (i + 1) * CH)
            ga, gb = gathers[b]
            ga.wait()
            gb.wait()
            a_v, b_v = ab[b], bb[b]

            def add_row(r, c1):
                def add_c(c, c2):
                    sl = pl.ds(c * 16, 16)
                    a_v[r, sl] = a_v[r, sl] + b_v[r, sl]
                    return c2
                return lax.fori_loop(0, D2 // 16, add_c, c1, unroll=8)

            lax.fori_loop(0, CH, add_row, 0)
            pltpu.sync_copy(a_v, out_hbm.at[pl.ds(base + i * CH, CH)])

    return k(ys, pos0, pos1)


def _ffn(gids, xs, W1b, b1r, W2b, b2r, ws):
    P, D = xs.shape
    E, _, H = W1b.shape
    n_m = P // _BM

    def kern(g_ref, xs_ref, w1_ref, b1_ref, w2_ref, b2_ref, ws_ref, o_ref):
        xb = xs_ref[...].astype(jnp.bfloat16)
        h = jnp.dot(xb, w1_ref[0], preferred_element_type=jnp.float32)
        h = jnp.maximum(h + b1_ref[0], 0.0).astype(jnp.bfloat16)
        acc = jnp.dot(h, w2_ref[0], preferred_element_type=jnp.float32)
        o_ref[...] = (acc + b2_ref[0]) * ws_ref[...]

    grid_spec = pltpu.PrefetchScalarGridSpec(
        num_scalar_prefetch=1,
        grid=(n_m,),
        in_specs=[
            pl.BlockSpec((_BM, D), lambda i, g: (i, 0)),
            pl.BlockSpec((1, D, H), lambda i, g: (g[i], 0, 0)),
            pl.BlockSpec((1, 1, H), lambda i, g: (g[i], 0, 0)),
            pl.BlockSpec((1, H, D), lambda i, g: (g[i], 0, 0)),
            pl.BlockSpec((1, 1, D), lambda i, g: (g[i], 0, 0)),
            pl.BlockSpec((_BM, 1), lambda i, g: (i, 0)),
        ],
        out_specs=pl.BlockSpec((_BM, D), lambda i, g: (i, 0)),
    )
    return pl.pallas_call(
        kern,
        grid_spec=grid_spec,
        out_shape=jax.ShapeDtypeStruct((P, D), jnp.float32),
        compiler_params=pltpu.CompilerParams(
            dimension_semantics=("arbitrary",)),
    )(gids, xs, W1b, b1r, W2b, b2r, ws)


def kernel(x, Wg, bg, Wn, bn, W1, b1, W2, b2):
    B, S, D = x.shape
    N = B * S
    E, _, H = W1.shape
    top_i, probs = _router(x, Wg, bg, Wn, bn)
    src_token, ws, gids, pos, P = _dispatch(top_i, probs, N)

    xs = _sc_gather(x.reshape(N, D), src_token)
    ys = _ffn(gids, xs,
              W1.astype(jnp.bfloat16), b1.reshape(E, 1, H),
              W2.astype(jnp.bfloat16), b2.reshape(E, 1, D),
              ws.reshape(P, 1))
    out = _sc_combine(ys, pos[:, 0], pos[:, 1])
    return out.reshape(B, S, D)


# speedup vs baseline: 1.0952x; 1.0453x over previous
import functools

import jax
import jax.numpy as jnp
from jax import lax
from jax.experimental import pallas as pl
from jax.experimental.pallas import tpu as pltpu
from jax.experimental.pallas import tpu_sc as plsc

_NUM_EXPERTS = 8
_TOP_K = 2
_BM = 128


def _router(x, Wg, bg, Wn, bn):
    logits = x @ Wg + bg
    safe_noise = jax.nn.softplus(x @ Wn + bn)
    eps = jax.random.normal(jax.random.key(42), logits.shape, dtype=logits.dtype)
    noisy_logits = logits + eps * safe_noise
    top_v, top_i = lax.top_k(noisy_logits, _TOP_K)
    probs = jax.nn.softmax(top_v, axis=-1)
    return top_i, probs


def _dispatch(top_i, probs, n_tokens):
    E, BM, K = _NUM_EXPERTS, _BM, _TOP_K
    A = n_tokens * K
    P = A + E * BM

    flat_e = top_i.reshape(-1).astype(jnp.int32)
    oh = jax.nn.one_hot(flat_e, E, dtype=jnp.int32)
    csum = jnp.cumsum(oh, axis=0)
    counts = csum[-1]
    rank = jnp.take_along_axis(csum, flat_e[:, None], axis=1)[:, 0] - 1
    padded = ((counts + BM - 1) // BM) * BM
    pstart = jnp.concatenate([jnp.zeros((1,), jnp.int32),
                              jnp.cumsum(padded)[:-1].astype(jnp.int32)])
    dst = pstart[flat_e] + rank
    src_assign = jnp.zeros((P,), jnp.int32).at[dst].set(
        jnp.arange(A, dtype=jnp.int32))
    valid = jnp.zeros((P,), jnp.bool_).at[dst].set(True)
    src_token = src_assign // K
    ws = jnp.where(valid, probs.reshape(-1)[src_assign], 0.0)
    n_m = P // BM
    gids = (jnp.searchsorted(pstart,
                             jnp.arange(n_m, dtype=jnp.int32) * BM,
                             side="right").astype(jnp.int32) - 1)
    pos = dst.reshape(n_tokens, K)
    return src_token, ws, gids, pos, P


def _sc_gather(table, idx):
    P = idx.shape[0]
    D2 = table.shape[1]
    info = plsc.get_sparse_core_info()
    NW = info.num_cores * info.num_subcores
    rows_per_w = P // NW
    CH = next(c for c in (48, 32, 16, 8) if rows_per_w % c == 0)
    n_ch = rows_per_w // CH
    mesh = plsc.VectorSubcoreMesh(core_axis_name="c", subcore_axis_name="s")

    @functools.partial(
        pl.kernel,
        mesh=mesh,
        out_type=jax.ShapeDtypeStruct((P, D2), jnp.float32),
        scratch_types=[
            pltpu.VMEM((CH,), jnp.int32),
            pltpu.VMEM((CH,), jnp.int32),
            pltpu.VMEM((CH, D2), jnp.float32),
            pltpu.VMEM((CH, D2), jnp.float32),
            pltpu.SemaphoreType.DMA,
            pltpu.SemaphoreType.DMA,
        ],
    )
    def k(table_hbm, idx_hbm, out_hbm, idx0, idx1, buf0, buf1, sg0, sg1):
        wid = lax.axis_index("s") * info.num_cores + lax.axis_index("c")
        base = wid * rows_per_w
        idxb, bufb, sg = [idx0, idx1], [buf0, buf1], [sg0, sg1]

        pltpu.sync_copy(idx_hbm.at[pl.ds(base, CH)], idx0)
        gathers = [pltpu.async_copy(table_hbm.at[idx0], buf0, sg0), None]
        for i in range(n_ch):
            b = i & 1
            nb = 1 - b
            if i + 1 < n_ch:
                off = base + (i + 1) * CH
                pltpu.sync_copy(idx_hbm.at[pl.ds(off, CH)], idxb[nb])
                gathers[nb] = pltpu.async_copy(
                    table_hbm.at[idxb[nb]], bufb[nb], sg[nb])
            gathers[b].wait()
            pltpu.sync_copy(bufb[b], out_hbm.at[pl.ds(base + i * CH, CH)])

    return k(table, idx)


def _sc_combine(ys, pos0, pos1):
    N = pos0.shape[0]
    D2 = ys.shape[1]
    info = plsc.get_sparse_core_info()
    NW = info.num_cores * info.num_subcores
    rows_per_w = N // NW
    CH = 16
    n_ch = rows_per_w // CH
    assert n_ch * CH == rows_per_w
    mesh = plsc.VectorSubcoreMesh(core_axis_name="c", subcore_axis_name="s")

    @functools.partial(
        pl.kernel,
        mesh=mesh,
        out_type=jax.ShapeDtypeStruct((N, D2), jnp.float32),
        scratch_types=[
            pltpu.VMEM((CH,), jnp.int32),
            pltpu.VMEM((CH,), jnp.int32),
            pltpu.VMEM((CH, D2), jnp.float32),
            pltpu.VMEM((CH, D2), jnp.float32),
            pltpu.VMEM((CH, D2), jnp.float32),
            pltpu.VMEM((CH, D2), jnp.float32),
            pltpu.SemaphoreType.DMA,
            pltpu.SemaphoreType.DMA,
            pltpu.SemaphoreType.DMA,
            pltpu.SemaphoreType.DMA,
        ],
    )
    def k(ys_hbm, p0_hbm, p1_hbm, out_hbm,
          i0, i1, a0, a1, b0, b1, sa0, sa1, sb0, sb1):
        wid = lax.axis_index("s") * info.num_cores + lax.axis_index("c")
        base = wid * rows_per_w
        idxb, ab, bb = [i0, i1], [a0, a1], [b0, b1]
        sa, sb = [sa0, sa1], [sb0, sb1]

        def issue(slot, off):
            pltpu.sync_copy(p0_hbm.at[pl.ds(off, CH)], idxb[slot])
            ga = pltpu.async_copy(ys_hbm.at[idxb[slot]], ab[slot], sa[slot])
            pltpu.sync_copy(p1_hbm.at[pl.ds(off, CH)], idxb[slot])
            gb = pltpu.async_copy(ys_hbm.at[idxb[slot]], bb[slot], sb[slot])
            return ga, gb

        gathers = [issue(0, base), None]
        for i in range(n_ch):
            b = i & 1
            nb = 1 - b
            if i + 1 < n_ch:
                gathers[nb] = issue(nb, base + (i + 1) * CH)
            ga, gb = gathers[b]
            ga.wait()
            gb.wait()
            a_v, b_v = ab[b], bb[b]

            def add_row(r, c1):
                def add_c(c, c2):
                    sl = pl.ds(c * 16, 16)
                    a_v[r, sl] = a_v[r, sl] + b_v[r, sl]
                    return c2
                return lax.fori_loop(0, D2 // 16, add_c, c1, unroll=8)

            lax.fori_loop(0, CH, add_row, 0)
            pltpu.sync_copy(a_v, out_hbm.at[pl.ds(base + i * CH, CH)])

    return k(ys, pos0, pos1)


def _ffn(gids, xs, W1b, b1r, W2b, b2r, ws):
    P, D = xs.shape
    E, _, H = W1b.shape
    n_m = P // _BM

    def kern(g_ref, xs_ref, w1_ref, b1_ref, w2_ref, b2_ref, ws_ref, o_ref):
        xb = xs_ref[...].astype(jnp.bfloat16)
        h = jnp.dot(xb, w1_ref[0], preferred_element_type=jnp.float32)
        h = jnp.maximum(h + b1_ref[0], 0.0).astype(jnp.bfloat16)
        acc = jnp.dot(h, w2_ref[0], preferred_element_type=jnp.float32)
        o_ref[...] = (acc + b2_ref[0]) * ws_ref[...]

    grid_spec = pltpu.PrefetchScalarGridSpec(
        num_scalar_prefetch=1,
        grid=(n_m,),
        in_specs=[
            pl.BlockSpec((_BM, D), lambda i, g: (i, 0)),
            pl.BlockSpec((1, D, H), lambda i, g: (g[i], 0, 0)),
            pl.BlockSpec((1, 1, H), lambda i, g: (g[i], 0, 0)),
            pl.BlockSpec((1, H, D), lambda i, g: (g[i], 0, 0)),
            pl.BlockSpec((1, 1, D), lambda i, g: (g[i], 0, 0)),
            pl.BlockSpec((_BM, 1), lambda i, g: (i, 0)),
        ],
        out_specs=pl.BlockSpec((_BM, D), lambda i, g: (i, 0)),
    )
    return pl.pallas_call(
        kern,
        grid_spec=grid_spec,
        out_shape=jax.ShapeDtypeStruct((P, D), jnp.float32),
        compiler_params=pltpu.CompilerParams(
            dimension_semantics=("arbitrary",)),
    )(gids, xs, W1b, b1r, W2b, b2r, ws)


def kernel(x, Wg, bg, Wn, bn, W1, b1, W2, b2):
    B, S, D = x.shape
    N = B * S
    E, _, H = W1.shape
    top_i, probs = _router(x, Wg, bg, Wn, bn)
    src_token, ws, gids, pos, P = _dispatch(top_i, probs, N)

    xs = _sc_gather(x.reshape(N, D), src_token)
    ys = _ffn(gids, xs,
              W1.astype(jnp.bfloat16), b1.reshape(E, 1, H),
              W2.astype(jnp.bfloat16), b2.reshape(E, 1, D),
              ws.reshape(P, 1))
    out = _sc_combine(ys, pos[:, 0], pos[:, 1])
    return out.reshape(B, S, D)
